# Initial kernel scaffold; baseline (speedup 1.0000x reference)
import jax
import jax.numpy as jnp
from jax.experimental import pallas as pl


def kernel(x, pos, edge_index, batch, W1l, b1l, W1r, W2l, b2l, W2r, Wlin, blin, Wout, bout):
    raise NotImplementedError("write your pallas kernel here")


# speedup vs baseline: 5.5554x; 5.5554x over previous
import functools

import jax
import jax.numpy as jnp
from jax import lax
from jax.experimental import pallas as pl
from jax.experimental.pallas import tpu as pltpu
from jax.experimental.pallas import tpu_sc as plsc

N = 10000
E = 320000
D = 128
POS = 3
POSP = 8
CONV = 128
LIN = 128
OUT = 10
G = 64

NC = 2
NS = 16
NW = NC * NS
EPW = E // NW
K = 80
NCH = EPW // K
IGB = 25
NIG = NCH // IGB
NP = 10240
ZCH = 32
NZT = NP // ZCH // NS
WCH = 128
NWT = NP // WCH // NS
CW = 16

RB = 1000
NB = N // RB

assert E == NW * NCH * K and NCH == NIG * IGB
assert NP == NZT * ZCH * NS == NWT * WCH * NS and NP >= N


def _make_agg(with_cnt):
    mesh = plsc.VectorSubcoreMesh(core_axis_name="c", subcore_axis_name="s")
    out_type = [jax.ShapeDtypeStruct((NC, NP, CONV), jnp.float32)]
    scratch = [
        pltpu.VMEM((IGB, K), jnp.int32),
        pltpu.VMEM((IGB, K), jnp.int32),
        pltpu.VMEM((K, CONV), jnp.float32),
        pltpu.VMEM_SHARED((NP, CONV), jnp.float32),
        pltpu.SemaphoreType.DMA,
    ]
    if with_cnt:
        out_type.append(jax.ShapeDtypeStruct((NC, NP, CONV), jnp.float32))
        scratch.append(pltpu.VMEM((K, CONV), jnp.float32))

    def body(z128_hbm, src_hbm, dst_hbm, g_hbm, *rest):
        if with_cnt:
            (a_hbm, c_hbm, src_v, dst_v, rows_v, acc_sh, sem, ones_v) = rest
        else:
            (a_hbm, src_v, dst_v, rows_v, acc_sh, sem) = rest
        c = lax.axis_index("c")
        s = lax.axis_index("s")
        w = c * NS + s
        row0 = s * (NP // NS)

        def _zero_acc():
            pltpu.sync_copy(z128_hbm.at[pl.ds(row0, NP // NS)],
                            acc_sh.at[pl.ds(row0, NP // NS)])

        def _writeback(dst_hbm_ref):
            @pl.loop(0, NWT)
            def _wb(t):
                pltpu.sync_copy(acc_sh.at[pl.ds(row0 + t * WCH, WCH)],
                                dst_hbm_ref.at[c, pl.ds(row0 + t * WCH, WCH)])

        _zero_acc()
        plsc.subcore_barrier()

        @pl.loop(0, NIG)
        def _grp(gidx):
            pltpu.sync_copy(src_hbm.at[w, gidx], src_v)
            pltpu.sync_copy(dst_hbm.at[w, gidx], dst_v)

            @pl.loop(0, IGB)
            def _step(j):
                pltpu.async_copy(g_hbm.at[src_v.at[j]], rows_v, sem).wait()
                pltpu.sync_copy(rows_v, acc_sh.at[dst_v.at[j]], add=True)

        plsc.subcore_barrier()
        _writeback(a_hbm)

        if with_cnt:
            @pl.loop(0, K)
            def _ob(i):
                @pl.loop(0, CONV // 16)
                def _obj(j):
                    ones_v[i, pl.ds(j * 16, 16)] = jnp.ones((16,), jnp.float32)

            plsc.subcore_barrier()
            _zero_acc()
            plsc.subcore_barrier()

            @pl.loop(0, NIG)
            def _cgrp(gidx):
                pltpu.sync_copy(dst_hbm.at[w, gidx], dst_v)

                @pl.loop(0, IGB)
                def _cstep(j):
                    pltpu.sync_copy(ones_v, acc_sh.at[dst_v.at[j]], add=True)

            plsc.subcore_barrier()
            _writeback(c_hbm)

    return pl.kernel(body, out_type=tuple(out_type), mesh=mesh,
                     scratch_types=scratch)


_agg_cnt = _make_agg(True)
_agg = _make_agg(False)


def _proj_body(x_ref, p_ref, wlx_ref, wlp_ref, wrx_ref, wrp_ref, b_ref,
               g_ref, r_ref):
    xb = x_ref[...]
    pb = p_ref[...]
    dot = functools.partial(jnp.dot, preferred_element_type=jnp.float32)
    g_ref[...] = dot(xb, wlx_ref[...]) + dot(pb, wlp_ref[...])
    r_ref[...] = dot(xb, wrx_ref[...]) + dot(pb, wrp_ref[...]) + b_ref[...]


def _proj(x, posp, wlx, wlp, wrx, wrp, b1):
    full = lambda shape: pl.BlockSpec(shape, lambda i: (0, 0))
    return pl.pallas_call(
        _proj_body,
        grid=(NB,),
        in_specs=[
            pl.BlockSpec((RB, D), lambda i: (i, 0)),
            pl.BlockSpec((RB, POSP), lambda i: (i, 0)),
            full((D, CONV)), full((POSP, CONV)),
            full((D, CONV)), full((POSP, CONV)),
            full((1, CONV)),
        ],
        out_specs=[pl.BlockSpec((RB, CONV), lambda i: (i, 0))] * 2,
        out_shape=[jax.ShapeDtypeStruct((N, CONV), jnp.float32)] * 2,
    )(x, posp, wlx, wlp, wrx, wrp, b1)


def _mid_body(a0_ref, a1_ref, c0_ref, c1_ref, r0_ref, w2l_ref, w2r_ref,
              b2_ref, g1_ref, r1_ref):
    a = a0_ref[0] + a1_ref[0]
    cnt = c0_ref[0][:, 0:1] + c1_ref[0][:, 0:1]
    h1 = jnp.maximum(a / jnp.maximum(cnt, 1.0) + r0_ref[...], 0.0)
    dot = functools.partial(jnp.dot, preferred_element_type=jnp.float32)
    g1_ref[...] = dot(h1, w2l_ref[...])
    r1_ref[...] = dot(h1, w2r_ref[...]) + b2_ref[...]


def _mid(A, C, r0, w2l, w2r, b2):
    full = lambda shape: pl.BlockSpec(shape, lambda i: (0, 0))
    return pl.pallas_call(
        _mid_body,
        grid=(NB,),
        in_specs=[
            pl.BlockSpec((1, RB, CONV), lambda i: (0, i, 0)),
            pl.BlockSpec((1, RB, CONV), lambda i: (1, i, 0)),
            pl.BlockSpec((1, RB, CONV), lambda i: (0, i, 0)),
            pl.BlockSpec((1, RB, CONV), lambda i: (1, i, 0)),
            pl.BlockSpec((RB, CONV), lambda i: (i, 0)),
            full((CONV, CONV)), full((CONV, CONV)), full((1, CONV)),
        ],
        out_specs=[pl.BlockSpec((RB, CONV), lambda i: (i, 0))] * 2,
        out_shape=[jax.ShapeDtypeStruct((N, CONV), jnp.float32)] * 2,
    )(A, A, C, C, r0, w2l, w2r, b2)


def _fin_body(b0_ref, b1_ref, c0_ref, c1_ref, r1_ref, bat_ref, wlin_ref,
              blin_ref, wout_ref, bout_ref, y_ref, p_acc):
    i = pl.program_id(0)

    @pl.when(i == 0)
    def _():
        p_acc[...] = jnp.full((G, CONV), -jnp.inf, jnp.float32)

    a = b0_ref[0] + b1_ref[0]
    cnt = c0_ref[0][:, 0:1] + c1_ref[0][:, 0:1]
    h2 = jnp.maximum(a / jnp.maximum(cnt, 1.0) + r1_ref[...], 0.0)
    bcol = bat_ref[0]

    def _grp(gidx, _):
        gf = lax.convert_element_type(gidx, jnp.float32)
        cand = jnp.max(jnp.where(bcol == gf, h2, -jnp.inf), axis=0,
                       keepdims=True)
        cur = p_acc[pl.ds(gidx, 1), :]
        p_acc[pl.ds(gidx, 1), :] = jnp.maximum(cur, cand)
        return 0
    lax.fori_loop(0, G, _grp, 0)

    @pl.when(i == NB - 1)
    def _():
        p = p_acc[...]
        p = jnp.where(jnp.isfinite(p), p, 0.0)
        dot = functools.partial(jnp.dot, preferred_element_type=jnp.float32)
        z = dot(p, wlin_ref[...]) + blin_ref[...]
        y_ref[...] = dot(z, wout_ref[...]) + bout_ref[...]


def _fin(B, C, r1, batf, wlin, blin, wout, bout):
    full = lambda shape: pl.BlockSpec(shape, lambda i: (0, 0))
    return pl.pallas_call(
        _fin_body,
        grid=(NB,),
        in_specs=[
            pl.BlockSpec((1, RB, CONV), lambda i: (0, i, 0)),
            pl.BlockSpec((1, RB, CONV), lambda i: (1, i, 0)),
            pl.BlockSpec((1, RB, CONV), lambda i: (0, i, 0)),
            pl.BlockSpec((1, RB, CONV), lambda i: (1, i, 0)),
            pl.BlockSpec((RB, CONV), lambda i: (i, 0)),
            pl.BlockSpec((1, RB, 1), lambda i: (i, 0, 0)),
            full((CONV, LIN)), full((1, LIN)),
            full((LIN, OUT)), full((1, OUT)),
        ],
        out_specs=pl.BlockSpec((G, OUT), lambda i: (0, 0)),
        out_shape=jax.ShapeDtypeStruct((G, OUT), jnp.float32),
        scratch_shapes=[pltpu.VMEM((G, CONV), jnp.float32)],
    )(B, B, C, C, r1, batf, wlin, blin, wout, bout)


def kernel(x, pos, edge_index, batch, W1l, b1l, W1r, W2l, b2l, W2r,
           Wlin, blin, Wout, bout):
    src = edge_index[0].reshape(NW, NIG, IGB, K)
    dst = edge_index[1].reshape(NW, NIG, IGB, K)
    posp = jnp.pad(pos, ((0, 0), (0, POSP - POS)))
    wpad = lambda w: jnp.pad(w[D:], ((0, POSP - POS), (0, 0)))

    z128 = jnp.zeros((NP, CONV), jnp.float32)
    g0, r0 = _proj(x, posp, W1l[:D], wpad(W1l), W1r[:D], wpad(W1r),
                   b1l.reshape(1, CONV))
    A, C = _agg_cnt(z128, src, dst, g0)
    g1, r1 = _mid(A, C, r0, W2l, W2r, b2l.reshape(1, CONV))
    (B,) = _agg(z128, src, dst, g1)
    batf = batch.astype(jnp.float32).reshape(NB, RB, 1)
    return _fin(B, C, r1, batf, Wlin, blin.reshape(1, LIN),
                Wout, bout.reshape(1, OUT))


# speedup vs baseline: 6.4479x; 1.1607x over previous
import functools

import jax
import jax.numpy as jnp
from jax import lax
from jax.experimental import pallas as pl
from jax.experimental.pallas import tpu as pltpu
from jax.experimental.pallas import tpu_sc as plsc

N = 10000
E = 320000
D = 128
POS = 3
POSP = 8
CONV = 128
LIN = 128
OUT = 10
G = 64

NC = 2
NS = 16
NW = NC * NS
EPW = E // NW
K = 80
NCH = EPW // K
IGB = 25
NIG = NCH // IGB
NP = 10240
ZCH = 32
NZT = NP // ZCH // NS
WCH = 128
NWT = NP // WCH // NS
CW = 16

RB = 1000
NB = N // RB

assert E == NW * NCH * K and NCH == NIG * IGB
assert NP == NZT * ZCH * NS == NWT * WCH * NS and NP >= N


def _make_agg(with_cnt):
    mesh = plsc.VectorSubcoreMesh(core_axis_name="c", subcore_axis_name="s")
    out_type = [jax.ShapeDtypeStruct((NC, NP, CONV), jnp.float32)]
    scratch = [
        pltpu.VMEM((IGB, K), jnp.int32),
        pltpu.VMEM((IGB, K), jnp.int32),
        pltpu.VMEM((K, CONV), jnp.float32),
        pltpu.VMEM((K, CONV), jnp.float32),
        pltpu.VMEM_SHARED((NP, CONV), jnp.float32),
        pltpu.SemaphoreType.DMA,
        pltpu.SemaphoreType.DMA,
    ]
    if with_cnt:
        out_type.append(jax.ShapeDtypeStruct((NC, NP, CONV), jnp.float32))
        scratch.append(pltpu.VMEM((K, CONV), jnp.float32))

    def body(z128_hbm, src_hbm, dst_hbm, g_hbm, *rest):
        if with_cnt:
            (a_hbm, c_hbm, src_v, dst_v, rows0, rows1, acc_sh, sem0, sem1,
             ones_v) = rest
        else:
            (a_hbm, src_v, dst_v, rows0, rows1, acc_sh, sem0, sem1) = rest
        c = lax.axis_index("c")
        s = lax.axis_index("s")
        w = c * NS + s
        row0 = s * (NP // NS)

        def _zero_acc():
            pltpu.sync_copy(z128_hbm.at[pl.ds(row0, NP // NS)],
                            acc_sh.at[pl.ds(row0, NP // NS)])

        def _writeback(dst_hbm_ref):
            @pl.loop(0, NWT)
            def _wb(t):
                pltpu.sync_copy(acc_sh.at[pl.ds(row0 + t * WCH, WCH)],
                                dst_hbm_ref.at[c, pl.ds(row0 + t * WCH, WCH)])

        _zero_acc()
        plsc.subcore_barrier()

        @pl.loop(0, NIG)
        def _grp(gidx):
            pltpu.sync_copy(src_hbm.at[w, gidx], src_v)
            pltpu.sync_copy(dst_hbm.at[w, gidx], dst_v)
            pltpu.async_copy(g_hbm.at[src_v.at[0]], rows0, sem0)

            @pl.loop(0, (IGB - 1) // 2)
            def _pipe(jj):
                j0 = 2 * jj
                pltpu.make_async_copy(g_hbm.at[src_v.at[0]], rows0, sem0).wait()
                pltpu.async_copy(g_hbm.at[src_v.at[j0 + 1]], rows1, sem1)
                pltpu.sync_copy(rows0, acc_sh.at[dst_v.at[j0]], add=True)
                pltpu.make_async_copy(g_hbm.at[src_v.at[0]], rows1, sem1).wait()
                pltpu.async_copy(g_hbm.at[src_v.at[j0 + 2]], rows0, sem0)
                pltpu.sync_copy(rows1, acc_sh.at[dst_v.at[j0 + 1]], add=True)

            pltpu.make_async_copy(g_hbm.at[src_v.at[0]], rows0, sem0).wait()
            pltpu.sync_copy(rows0, acc_sh.at[dst_v.at[IGB - 1]], add=True)

        plsc.subcore_barrier()
        _writeback(a_hbm)

        if with_cnt:
            @pl.loop(0, K)
            def _ob(i):
                @pl.loop(0, CONV // 16)
                def _obj(j):
                    ones_v[i, pl.ds(j * 16, 16)] = jnp.ones((16,), jnp.float32)

            plsc.subcore_barrier()
            _zero_acc()
            plsc.subcore_barrier()

            @pl.loop(0, NIG)
            def _cgrp(gidx):
                pltpu.sync_copy(dst_hbm.at[w, gidx], dst_v)

                @pl.loop(0, IGB)
                def _cstep(j):
                    pltpu.sync_copy(ones_v, acc_sh.at[dst_v.at[j]], add=True)

            plsc.subcore_barrier()
            _writeback(c_hbm)

    return pl.kernel(body, out_type=tuple(out_type), mesh=mesh,
                     scratch_types=scratch)


_agg_cnt = _make_agg(True)
_agg = _make_agg(False)


def _proj_body(x_ref, p_ref, wlx_ref, wlp_ref, wrx_ref, wrp_ref, b_ref,
               g_ref, r_ref):
    xb = x_ref[...]
    pb = p_ref[...]
    dot = functools.partial(jnp.dot, preferred_element_type=jnp.float32)
    g_ref[...] = dot(xb, wlx_ref[...]) + dot(pb, wlp_ref[...])
    r_ref[...] = dot(xb, wrx_ref[...]) + dot(pb, wrp_ref[...]) + b_ref[...]


def _proj(x, posp, wlx, wlp, wrx, wrp, b1):
    full = lambda shape: pl.BlockSpec(shape, lambda i: (0, 0))
    return pl.pallas_call(
        _proj_body,
        grid=(NB,),
        in_specs=[
            pl.BlockSpec((RB, D), lambda i: (i, 0)),
            pl.BlockSpec((RB, POSP), lambda i: (i, 0)),
            full((D, CONV)), full((POSP, CONV)),
            full((D, CONV)), full((POSP, CONV)),
            full((1, CONV)),
        ],
        out_specs=[pl.BlockSpec((RB, CONV), lambda i: (i, 0))] * 2,
        out_shape=[jax.ShapeDtypeStruct((N, CONV), jnp.float32)] * 2,
    )(x, posp, wlx, wlp, wrx, wrp, b1)


def _mid_body(a0_ref, a1_ref, c0_ref, c1_ref, r0_ref, w2l_ref, w2r_ref,
              b2_ref, g1_ref, r1_ref):
    a = a0_ref[0] + a1_ref[0]
    cnt = c0_ref[0][:, 0:1] + c1_ref[0][:, 0:1]
    h1 = jnp.maximum(a / jnp.maximum(cnt, 1.0) + r0_ref[...], 0.0)
    dot = functools.partial(jnp.dot, preferred_element_type=jnp.float32)
    g1_ref[...] = dot(h1, w2l_ref[...])
    r1_ref[...] = dot(h1, w2r_ref[...]) + b2_ref[...]


def _mid(A, C, r0, w2l, w2r, b2):
    full = lambda shape: pl.BlockSpec(shape, lambda i: (0, 0))
    return pl.pallas_call(
        _mid_body,
        grid=(NB,),
        in_specs=[
            pl.BlockSpec((1, RB, CONV), lambda i: (0, i, 0)),
            pl.BlockSpec((1, RB, CONV), lambda i: (1, i, 0)),
            pl.BlockSpec((1, RB, CONV), lambda i: (0, i, 0)),
            pl.BlockSpec((1, RB, CONV), lambda i: (1, i, 0)),
            pl.BlockSpec((RB, CONV), lambda i: (i, 0)),
            full((CONV, CONV)), full((CONV, CONV)), full((1, CONV)),
        ],
        out_specs=[pl.BlockSpec((RB, CONV), lambda i: (i, 0))] * 2,
        out_shape=[jax.ShapeDtypeStruct((N, CONV), jnp.float32)] * 2,
    )(A, A, C, C, r0, w2l, w2r, b2)


def _fin_body(b0_ref, b1_ref, c0_ref, c1_ref, r1_ref, bat_ref, wlin_ref,
              blin_ref, wout_ref, bout_ref, y_ref, p_acc):
    i = pl.program_id(0)

    @pl.when(i == 0)
    def _():
        p_acc[...] = jnp.full((G, CONV), -jnp.inf, jnp.float32)

    a = b0_ref[0] + b1_ref[0]
    cnt = c0_ref[0][:, 0:1] + c1_ref[0][:, 0:1]
    h2 = jnp.maximum(a / jnp.maximum(cnt, 1.0) + r1_ref[...], 0.0)
    bcol = bat_ref[0]

    def _grp(gidx, _):
        gf = lax.convert_element_type(gidx, jnp.float32)
        cand = jnp.max(jnp.where(bcol == gf, h2, -jnp.inf), axis=0,
                       keepdims=True)
        cur = p_acc[pl.ds(gidx, 1), :]
        p_acc[pl.ds(gidx, 1), :] = jnp.maximum(cur, cand)
        return 0
    lax.fori_loop(0, G, _grp, 0)

    @pl.when(i == NB - 1)
    def _():
        p = p_acc[...]
        p = jnp.where(jnp.isfinite(p), p, 0.0)
        dot = functools.partial(jnp.dot, preferred_element_type=jnp.float32)
        z = dot(p, wlin_ref[...]) + blin_ref[...]
        y_ref[...] = dot(z, wout_ref[...]) + bout_ref[...]


def _fin(B, C, r1, batf, wlin, blin, wout, bout):
    full = lambda shape: pl.BlockSpec(shape, lambda i: (0, 0))
    return pl.pallas_call(
        _fin_body,
        grid=(NB,),
        in_specs=[
            pl.BlockSpec((1, RB, CONV), lambda i: (0, i, 0)),
            pl.BlockSpec((1, RB, CONV), lambda i: (1, i, 0)),
            pl.BlockSpec((1, RB, CONV), lambda i: (0, i, 0)),
            pl.BlockSpec((1, RB, CONV), lambda i: (1, i, 0)),
            pl.BlockSpec((RB, CONV), lambda i: (i, 0)),
            pl.BlockSpec((1, RB, 1), lambda i: (i, 0, 0)),
            full((CONV, LIN)), full((1, LIN)),
            full((LIN, OUT)), full((1, OUT)),
        ],
        out_specs=pl.BlockSpec((G, OUT), lambda i: (0, 0)),
        out_shape=jax.ShapeDtypeStruct((G, OUT), jnp.float32),
        scratch_shapes=[pltpu.VMEM((G, CONV), jnp.float32)],
    )(B, B, C, C, r1, batf, wlin, blin, wout, bout)


def kernel(x, pos, edge_index, batch, W1l, b1l, W1r, W2l, b2l, W2r,
           Wlin, blin, Wout, bout):
    src = edge_index[0].reshape(NW, NIG, IGB, K)
    dst = edge_index[1].reshape(NW, NIG, IGB, K)
    posp = jnp.pad(pos, ((0, 0), (0, POSP - POS)))
    wpad = lambda w: jnp.pad(w[D:], ((0, POSP - POS), (0, 0)))

    z128 = jnp.zeros((NP, CONV), jnp.float32)
    g0, r0 = _proj(x, posp, W1l[:D], wpad(W1l), W1r[:D], wpad(W1r),
                   b1l.reshape(1, CONV))
    A, C = _agg_cnt(z128, src, dst, g0)
    g1, r1 = _mid(A, C, r0, W2l, W2r, b2l.reshape(1, CONV))
    (B,) = _agg(z128, src, dst, g1)
    batf = batch.astype(jnp.float32).reshape(NB, RB, 1)
    return _fin(B, C, r1, batf, Wlin, blin.reshape(1, LIN),
                Wout, bout.reshape(1, OUT))
